# Initial kernel scaffold; baseline (speedup 1.0000x reference)
import jax
import jax.numpy as jnp
from jax.experimental import pallas as pl


def kernel(x, weights, targets):
    raise NotImplementedError("write your pallas kernel here")


# speedup vs baseline: 178.6846x; 178.6846x over previous
import functools

import jax
import jax.numpy as jnp
from jax import lax
from jax.experimental import pallas as pl
from jax.experimental.pallas import tpu as pltpu
from jax.experimental.pallas import tpu_sc as plsc

IN_FEATURES = 4096
OUT_FEATURES = 16384
FAN_OUT = 4096

NUM_CORES = 2
NUM_SUBCORES = 16
LANES = 16
NUM_WORKERS = NUM_CORES * NUM_SUBCORES
ROWS_PER_WORKER = IN_FEATURES // NUM_WORKERS
CHUNK_ROWS = 4
NUM_CHUNKS = ROWS_PER_WORKER // CHUNK_ROWS
NBUF = 2
VECS_PER_ROW = FAN_OUT // LANES


def _sc_body(x_hbm, w_hbm, t_hbm, out_hbm,
             x_v, w_buf, t_buf, acc,
             sem_w0, sem_w1, sem_t0, sem_t1):
  wid = lax.axis_index("s") * NUM_CORES + lax.axis_index("c")
  row0 = wid * ROWS_PER_WORKER

  w_sems = (sem_w0, sem_w1)
  t_sems = (sem_t0, sem_t1)

  pltpu.sync_copy(x_hbm.at[pl.ds(row0, ROWS_PER_WORKER)],
                  x_v.at[pl.ds(0, ROWS_PER_WORKER)])

  zeros16 = jnp.zeros((LANES,), jnp.float32)

  @plsc.parallel_loop(0, OUT_FEATURES // LANES, unroll=8)
  def _zero(i):
    acc[pl.ds(i * LANES, LANES)] = zeros16

  def start_fill(slot, c):
    row = row0 + c * CHUNK_ROWS
    pltpu.async_copy(w_hbm.at[pl.ds(row, CHUNK_ROWS), :], w_buf.at[slot],
                     w_sems[slot])
    pltpu.async_copy(t_hbm.at[pl.ds(row, CHUNK_ROWS), :], t_buf.at[slot],
                     t_sems[slot])

  def wait_fill(slot):
    pltpu.make_async_copy(w_hbm.at[pl.ds(0, CHUNK_ROWS), :], w_buf.at[slot],
                          w_sems[slot]).wait()
    pltpu.make_async_copy(t_hbm.at[pl.ds(0, CHUNK_ROWS), :], t_buf.at[slot],
                          t_sems[slot]).wait()

  start_fill(0, 0)
  start_fill(1, 1)

  @pl.loop(0, NUM_CHUNKS, step=NBUF)
  def _chunks(g):
    for slot in range(NBUF):
      c = g + slot
      wait_fill(slot)
      for r in range(CHUNK_ROWS):
        win = x_v[pl.ds(c * CHUNK_ROWS + r, LANES)]
        xvec = jnp.full((LANES,), win[0], jnp.float32)

        @plsc.parallel_loop(0, VECS_PER_ROW, unroll=8)
        def _vecs(k):
          w16 = w_buf[slot, r, pl.ds(k * LANES, LANES)]
          t16 = t_buf[slot, r, pl.ds(k * LANES, LANES)]
          plsc.addupdate_scatter(acc, [t16], w16 * xvec)

      @pl.when(c + NBUF < NUM_CHUNKS)
      def _():
        start_fill(slot, c + NBUF)

  pltpu.sync_copy(acc, out_hbm.at[wid])


@jax.jit
def _sc_scatter(x, weights, targets):
  mesh = plsc.VectorSubcoreMesh(core_axis_name="c", subcore_axis_name="s")
  kernel_fn = pl.kernel(
      _sc_body,
      out_type=jax.ShapeDtypeStruct((NUM_WORKERS, OUT_FEATURES), jnp.float32),
      mesh=mesh,
      scratch_types=[
          pltpu.VMEM((ROWS_PER_WORKER + LANES,), jnp.float32),
          pltpu.VMEM((NBUF, CHUNK_ROWS, FAN_OUT), jnp.float32),
          pltpu.VMEM((NBUF, CHUNK_ROWS, FAN_OUT), jnp.int32),
          pltpu.VMEM((OUT_FEATURES,), jnp.float32),
          pltpu.SemaphoreType.DMA,
          pltpu.SemaphoreType.DMA,
          pltpu.SemaphoreType.DMA,
          pltpu.SemaphoreType.DMA,
      ],
      compiler_params=pltpu.CompilerParams(needs_layout_passes=False),
  )
  return kernel_fn(x, weights, targets)


def _tc_sum_body(p_ref, o_ref):
  o_ref[...] = jnp.sum(p_ref[...], axis=0)


@jax.jit
def _tc_sum(partials):
  return pl.pallas_call(
      _tc_sum_body,
      out_shape=jax.ShapeDtypeStruct((OUT_FEATURES,), jnp.float32),
  )(partials)


def kernel(x, weights, targets):
  targets = targets.astype(jnp.int32)
  partials = _sc_scatter(x, weights, targets)
  return _tc_sum(partials)
